# baseline (device time: 115100 ns/iter reference)
import jax
import jax.numpy as jnp
from jax import lax
from jax.experimental import pallas as pl
from jax.experimental.pallas import tpu as pltpu

N_DEV = 4
M_CHUNK = 1024
N_HALF = 1024
N_HOPS = N_DEV - 1


def kernel(x, w_mat):
    def body(x_ref, w_ref, out_ref,
             part_r, part_l, recv_r, recv_l,
             send_r_sems, recv_r_sems, send_l_sems, recv_l_sems):
        my = lax.axis_index("i")
        right = (my + 1) % N_DEV
        left = (my + N_DEV - 1) % N_DEV

        barrier = pltpu.get_barrier_semaphore()
        for nbr in (left, right):
            pl.semaphore_signal(barrier, inc=1, device_id=(nbr,),
                                device_id_type=pl.DeviceIdType.MESH)
        pl.semaphore_wait(barrier, 2)

        for c in range(N_DEV):
            xc = x_ref[c * M_CHUNK:(c + 1) * M_CHUNK, :]
            pr = jnp.dot(xc, w_ref[:, :N_HALF],
                         preferred_element_type=jnp.float32)
            plf = jnp.dot(xc, w_ref[:, N_HALF:],
                          preferred_element_type=jnp.float32)
            part_r[c * M_CHUNK:(c + 1) * M_CHUNK, :] = pr.astype(jnp.bfloat16)
            part_l[c * M_CHUNK:(c + 1) * M_CHUNK, :] = plf.astype(jnp.bfloat16)

        def rows(c):
            return pl.ds(c * M_CHUNK, M_CHUNK)

        for h in range(N_HOPS):
            c_r = (my + N_DEV - 1 - h) % N_DEV
            c_l = (my + 1 + h) % N_DEV
            rdma_r = pltpu.make_async_remote_copy(
                src_ref=part_r.at[rows(c_r)],
                dst_ref=recv_r.at[h],
                send_sem=send_r_sems.at[h],
                recv_sem=recv_r_sems.at[h],
                device_id=(right,),
                device_id_type=pl.DeviceIdType.MESH,
            )
            rdma_l = pltpu.make_async_remote_copy(
                src_ref=part_l.at[rows(c_l)],
                dst_ref=recv_l.at[h],
                send_sem=send_l_sems.at[h],
                recv_sem=recv_l_sems.at[h],
                device_id=(left,),
                device_id_type=pl.DeviceIdType.MESH,
            )
            rdma_r.start()
            rdma_l.start()
            rdma_r.wait()
            rdma_l.wait()

            if h < N_HOPS - 1:
                nc_r = (my + N_DEV - 2 - h) % N_DEV
                nc_l = (my + 2 + h) % N_DEV
                part_r[rows(nc_r), :] = (
                    part_r[rows(nc_r), :].astype(jnp.float32)
                    + recv_r[h].astype(jnp.float32)).astype(jnp.bfloat16)
                part_l[rows(nc_l), :] = (
                    part_l[rows(nc_l), :].astype(jnp.float32)
                    + recv_l[h].astype(jnp.float32)).astype(jnp.bfloat16)
            else:
                out_ref[:, :N_HALF] = (
                    recv_r[h].astype(jnp.float32)
                    + part_r[rows(my), :].astype(jnp.float32))
                out_ref[:, N_HALF:] = (
                    recv_l[h].astype(jnp.float32)
                    + part_l[rows(my), :].astype(jnp.float32))

    return pl.pallas_call(
        body,
        out_shape=jax.ShapeDtypeStruct((M_CHUNK, 2 * N_HALF), jnp.float32),
        in_specs=[pl.BlockSpec(memory_space=pltpu.VMEM),
                  pl.BlockSpec(memory_space=pltpu.VMEM)],
        out_specs=pl.BlockSpec(memory_space=pltpu.VMEM),
        scratch_shapes=[
            pltpu.VMEM((N_DEV * M_CHUNK, N_HALF), jnp.bfloat16),
            pltpu.VMEM((N_DEV * M_CHUNK, N_HALF), jnp.bfloat16),
            pltpu.VMEM((N_HOPS, M_CHUNK, N_HALF), jnp.bfloat16),
            pltpu.VMEM((N_HOPS, M_CHUNK, N_HALF), jnp.bfloat16),
            pltpu.SemaphoreType.DMA((N_HOPS,)),
            pltpu.SemaphoreType.DMA((N_HOPS,)),
            pltpu.SemaphoreType.DMA((N_HOPS,)),
            pltpu.SemaphoreType.DMA((N_HOPS,)),
        ],
        compiler_params=pltpu.CompilerParams(
            collective_id=0,
            vmem_limit_bytes=100 * 1024 * 1024,
        ),
    )(x, w_mat)


# device time: 101248 ns/iter; 1.1368x vs baseline; 1.1368x over previous
import jax
import jax.numpy as jnp
from jax import lax
from jax.experimental import pallas as pl
from jax.experimental.pallas import tpu as pltpu

N_DEV = 4
M_CHUNK = 1024
N_HALF = 1024
N_HOPS = N_DEV - 1


def kernel(x, w_mat):
    def body(x_ref, w_ref, out_ref,
             part_r, part_l, recv_r, recv_l,
             send_r_sems, recv_r_sems, send_l_sems, recv_l_sems):
        my = lax.axis_index("i")
        right = (my + 1) % N_DEV
        left = (my + N_DEV - 1) % N_DEV

        barrier = pltpu.get_barrier_semaphore()
        for nbr in (left, right):
            pl.semaphore_signal(barrier, inc=1, device_id=(nbr,),
                                device_id_type=pl.DeviceIdType.MESH)
        pl.semaphore_wait(barrier, 2)

        def rows(c):
            return pl.ds(c * M_CHUNK, M_CHUNK)

        def xrows(c):
            return x_ref[rows(c), :]

        def dot_r(c):
            return jnp.dot(xrows(c), w_ref[:, :N_HALF],
                           preferred_element_type=jnp.float32)

        def dot_l(c):
            return jnp.dot(xrows(c), w_ref[:, N_HALF:],
                           preferred_element_type=jnp.float32)

        def make_rdma(h, c_r, c_l):
            rdma_r = pltpu.make_async_remote_copy(
                src_ref=part_r.at[rows(c_r)],
                dst_ref=recv_r.at[h],
                send_sem=send_r_sems.at[h],
                recv_sem=recv_r_sems.at[h],
                device_id=(right,),
                device_id_type=pl.DeviceIdType.MESH,
            )
            rdma_l = pltpu.make_async_remote_copy(
                src_ref=part_l.at[rows(c_l)],
                dst_ref=recv_l.at[h],
                send_sem=send_l_sems.at[h],
                recv_sem=recv_l_sems.at[h],
                device_id=(left,),
                device_id_type=pl.DeviceIdType.MESH,
            )
            return rdma_r, rdma_l

        hop_chunks = [((my + N_DEV - 1 - h) % N_DEV, (my + 1 + h) % N_DEV)
                      for h in range(N_HOPS)]

        c_r0, c_l0 = hop_chunks[0]
        part_r[rows(c_r0), :] = dot_r(c_r0).astype(jnp.bfloat16)
        part_l[rows(c_l0), :] = dot_l(c_l0).astype(jnp.bfloat16)
        rdmas = [make_rdma(0, c_r0, c_l0)]
        rdmas[0][0].start()
        rdmas[0][1].start()

        c_r1, c_l1 = hop_chunks[1]
        part_r[rows(c_r1), :] = dot_r(c_r1).astype(jnp.bfloat16)
        part_l[rows(c_l1), :] = dot_l(c_l1).astype(jnp.bfloat16)
        c_r2, c_l2 = hop_chunks[2]
        part_r[rows(c_r2), :] = dot_r(c_r2).astype(jnp.bfloat16)
        part_l[rows(c_l2), :] = dot_l(c_l2).astype(jnp.bfloat16)
        part_r[rows(my), :] = dot_r(my).astype(jnp.bfloat16)
        part_l[rows(my), :] = dot_l(my).astype(jnp.bfloat16)

        for h in range(N_HOPS):
            rdma_r, rdma_l = rdmas[h]
            rdma_r.wait_recv()
            rdma_l.wait_recv()
            if h < N_HOPS - 1:
                nc_r, nc_l = hop_chunks[h + 1]
                part_r[rows(nc_r), :] = (
                    part_r[rows(nc_r), :].astype(jnp.float32)
                    + recv_r[h].astype(jnp.float32)).astype(jnp.bfloat16)
                part_l[rows(nc_l), :] = (
                    part_l[rows(nc_l), :].astype(jnp.float32)
                    + recv_l[h].astype(jnp.float32)).astype(jnp.bfloat16)
                nxt = make_rdma(h + 1, nc_r, nc_l)
                nxt[0].start()
                nxt[1].start()
                rdmas.append(nxt)
            else:
                out_ref[:, :N_HALF] = (
                    recv_r[h].astype(jnp.float32)
                    + part_r[rows(my), :].astype(jnp.float32))
                out_ref[:, N_HALF:] = (
                    recv_l[h].astype(jnp.float32)
                    + part_l[rows(my), :].astype(jnp.float32))

        for rdma_r, rdma_l in rdmas:
            rdma_r.wait_send()
            rdma_l.wait_send()

    return pl.pallas_call(
        body,
        out_shape=jax.ShapeDtypeStruct((M_CHUNK, 2 * N_HALF), jnp.float32),
        in_specs=[pl.BlockSpec(memory_space=pltpu.VMEM),
                  pl.BlockSpec(memory_space=pltpu.VMEM)],
        out_specs=pl.BlockSpec(memory_space=pltpu.VMEM),
        scratch_shapes=[
            pltpu.VMEM((N_DEV * M_CHUNK, N_HALF), jnp.bfloat16),
            pltpu.VMEM((N_DEV * M_CHUNK, N_HALF), jnp.bfloat16),
            pltpu.VMEM((N_HOPS, M_CHUNK, N_HALF), jnp.bfloat16),
            pltpu.VMEM((N_HOPS, M_CHUNK, N_HALF), jnp.bfloat16),
            pltpu.SemaphoreType.DMA((N_HOPS,)),
            pltpu.SemaphoreType.DMA((N_HOPS,)),
            pltpu.SemaphoreType.DMA((N_HOPS,)),
            pltpu.SemaphoreType.DMA((N_HOPS,)),
        ],
        compiler_params=pltpu.CompilerParams(
            collective_id=0,
            vmem_limit_bytes=100 * 1024 * 1024,
        ),
    )(x, w_mat)


# device time: 95884 ns/iter; 1.2004x vs baseline; 1.0559x over previous
import jax
import jax.numpy as jnp
from jax import lax
from jax.experimental import pallas as pl
from jax.experimental.pallas import tpu as pltpu

N_DEV = 4
M_CHUNK = 1024
N_HALF = 1024
N_HOPS = N_DEV - 1
N_SUB = 2
SUB = M_CHUNK // N_SUB


def kernel(x, w_mat):
    def body(x_ref, w_ref, out_ref,
             part_r, part_l, recv_r, recv_l,
             send_r_sems, recv_r_sems, send_l_sems, recv_l_sems):
        my = lax.axis_index("i")
        right = (my + 1) % N_DEV
        left = (my + N_DEV - 1) % N_DEV

        barrier = pltpu.get_barrier_semaphore()
        for nbr in (left, right):
            pl.semaphore_signal(barrier, inc=1, device_id=(nbr,),
                                device_id_type=pl.DeviceIdType.MESH)
        pl.semaphore_wait(barrier, 2)

        def rows(c):
            return pl.ds(c * M_CHUNK, M_CHUNK)

        def sub_rows(c, s):
            return pl.ds(c * M_CHUNK + s * SUB, SUB)

        def dot_r(c):
            return jnp.dot(x_ref[rows(c), :], w_ref[:, :N_HALF],
                           preferred_element_type=jnp.float32)

        def dot_l(c):
            return jnp.dot(x_ref[rows(c), :], w_ref[:, N_HALF:],
                           preferred_element_type=jnp.float32)

        def make_rdma(h, s, c_r, c_l):
            rdma_r = pltpu.make_async_remote_copy(
                src_ref=part_r.at[sub_rows(c_r, s)],
                dst_ref=recv_r.at[sub_rows(h, s)],
                send_sem=send_r_sems.at[h, s],
                recv_sem=recv_r_sems.at[h, s],
                device_id=(right,),
                device_id_type=pl.DeviceIdType.MESH,
            )
            rdma_l = pltpu.make_async_remote_copy(
                src_ref=part_l.at[sub_rows(c_l, s)],
                dst_ref=recv_l.at[sub_rows(h, s)],
                send_sem=send_l_sems.at[h, s],
                recv_sem=recv_l_sems.at[h, s],
                device_id=(left,),
                device_id_type=pl.DeviceIdType.MESH,
            )
            return rdma_r, rdma_l

        hop_chunks = [((my + N_DEV - 1 - h) % N_DEV, (my + 1 + h) % N_DEV)
                      for h in range(N_HOPS)]

        c_r0, c_l0 = hop_chunks[0]
        part_r[rows(c_r0), :] = dot_r(c_r0).astype(jnp.bfloat16)
        part_l[rows(c_l0), :] = dot_l(c_l0).astype(jnp.bfloat16)
        rdmas = {}
        for s in range(N_SUB):
            rdmas[(0, s)] = make_rdma(0, s, c_r0, c_l0)
            rdmas[(0, s)][0].start()
            rdmas[(0, s)][1].start()

        c_r1, c_l1 = hop_chunks[1]
        part_r[rows(c_r1), :] = dot_r(c_r1).astype(jnp.bfloat16)
        part_l[rows(c_l1), :] = dot_l(c_l1).astype(jnp.bfloat16)
        c_r2, c_l2 = hop_chunks[2]
        part_r[rows(c_r2), :] = dot_r(c_r2).astype(jnp.bfloat16)
        part_l[rows(c_l2), :] = dot_l(c_l2).astype(jnp.bfloat16)
        part_r[rows(my), :] = dot_r(my).astype(jnp.bfloat16)
        part_l[rows(my), :] = dot_l(my).astype(jnp.bfloat16)

        for h in range(N_HOPS):
            for s in range(N_SUB):
                rdma_r, rdma_l = rdmas[(h, s)]
                if h < N_HOPS - 1:
                    nc_r, nc_l = hop_chunks[h + 1]
                    rdma_r.wait_recv()
                    part_r[sub_rows(nc_r, s), :] = (
                        part_r[sub_rows(nc_r, s), :].astype(jnp.float32)
                        + recv_r[sub_rows(h, s), :].astype(jnp.float32)
                    ).astype(jnp.bfloat16)
                    nxt = make_rdma(h + 1, s, nc_r, nc_l)
                    nxt[0].start()
                    rdma_l.wait_recv()
                    part_l[sub_rows(nc_l, s), :] = (
                        part_l[sub_rows(nc_l, s), :].astype(jnp.float32)
                        + recv_l[sub_rows(h, s), :].astype(jnp.float32)
                    ).astype(jnp.bfloat16)
                    nxt[1].start()
                    rdmas[(h + 1, s)] = nxt
                else:
                    rdma_r.wait_recv()
                    out_ref[pl.ds(s * SUB, SUB), :N_HALF] = (
                        recv_r[sub_rows(h, s), :].astype(jnp.float32)
                        + part_r[sub_rows(my, s), :].astype(jnp.float32))
                    rdma_l.wait_recv()
                    out_ref[pl.ds(s * SUB, SUB), N_HALF:] = (
                        recv_l[sub_rows(h, s), :].astype(jnp.float32)
                        + part_l[sub_rows(my, s), :].astype(jnp.float32))

        for rdma_r, rdma_l in rdmas.values():
            rdma_r.wait_send()
            rdma_l.wait_send()

    return pl.pallas_call(
        body,
        out_shape=jax.ShapeDtypeStruct((M_CHUNK, 2 * N_HALF), jnp.float32),
        in_specs=[pl.BlockSpec(memory_space=pltpu.VMEM),
                  pl.BlockSpec(memory_space=pltpu.VMEM)],
        out_specs=pl.BlockSpec(memory_space=pltpu.VMEM),
        scratch_shapes=[
            pltpu.VMEM((N_DEV * M_CHUNK, N_HALF), jnp.bfloat16),
            pltpu.VMEM((N_DEV * M_CHUNK, N_HALF), jnp.bfloat16),
            pltpu.VMEM((N_HOPS * M_CHUNK, N_HALF), jnp.bfloat16),
            pltpu.VMEM((N_HOPS * M_CHUNK, N_HALF), jnp.bfloat16),
            pltpu.SemaphoreType.DMA((N_HOPS, N_SUB)),
            pltpu.SemaphoreType.DMA((N_HOPS, N_SUB)),
            pltpu.SemaphoreType.DMA((N_HOPS, N_SUB)),
            pltpu.SemaphoreType.DMA((N_HOPS, N_SUB)),
        ],
        compiler_params=pltpu.CompilerParams(
            collective_id=0,
            vmem_limit_bytes=100 * 1024 * 1024,
        ),
    )(x, w_mat)


# device time: 95861 ns/iter; 1.2007x vs baseline; 1.0002x over previous
import jax
import jax.numpy as jnp
from jax import lax
from jax.experimental import pallas as pl
from jax.experimental.pallas import tpu as pltpu

N_DEV = 4
M_CHUNK = 1024
N_HALF = 1024
N_HOPS = N_DEV - 1
N_SUB = 4
SUB = M_CHUNK // N_SUB


def kernel(x, w_mat):
    def body(x_ref, w_ref, out_ref,
             part_r, part_l, recv_r, recv_l,
             send_r_sems, recv_r_sems, send_l_sems, recv_l_sems):
        my = lax.axis_index("i")
        right = (my + 1) % N_DEV
        left = (my + N_DEV - 1) % N_DEV

        barrier = pltpu.get_barrier_semaphore()
        for nbr in (left, right):
            pl.semaphore_signal(barrier, inc=1, device_id=(nbr,),
                                device_id_type=pl.DeviceIdType.MESH)
        pl.semaphore_wait(barrier, 2)

        def rows(c):
            return pl.ds(c * M_CHUNK, M_CHUNK)

        def sub_rows(c, s):
            return pl.ds(c * M_CHUNK + s * SUB, SUB)

        def dot_r(c):
            return jnp.dot(x_ref[rows(c), :], w_ref[:, :N_HALF],
                           preferred_element_type=jnp.float32)

        def dot_l(c):
            return jnp.dot(x_ref[rows(c), :], w_ref[:, N_HALF:],
                           preferred_element_type=jnp.float32)

        def make_rdma(h, s, c_r, c_l):
            rdma_r = pltpu.make_async_remote_copy(
                src_ref=part_r.at[sub_rows(c_r, s)],
                dst_ref=recv_r.at[sub_rows(h, s)],
                send_sem=send_r_sems.at[h, s],
                recv_sem=recv_r_sems.at[h, s],
                device_id=(right,),
                device_id_type=pl.DeviceIdType.MESH,
            )
            rdma_l = pltpu.make_async_remote_copy(
                src_ref=part_l.at[sub_rows(c_l, s)],
                dst_ref=recv_l.at[sub_rows(h, s)],
                send_sem=send_l_sems.at[h, s],
                recv_sem=recv_l_sems.at[h, s],
                device_id=(left,),
                device_id_type=pl.DeviceIdType.MESH,
            )
            return rdma_r, rdma_l

        hop_chunks = [((my + N_DEV - 1 - h) % N_DEV, (my + 1 + h) % N_DEV)
                      for h in range(N_HOPS)]

        c_r0, c_l0 = hop_chunks[0]
        part_r[rows(c_r0), :] = dot_r(c_r0).astype(jnp.bfloat16)
        part_l[rows(c_l0), :] = dot_l(c_l0).astype(jnp.bfloat16)
        rdmas = {}
        for s in range(N_SUB):
            rdmas[(0, s)] = make_rdma(0, s, c_r0, c_l0)
            rdmas[(0, s)][0].start()
            rdmas[(0, s)][1].start()

        c_r1, c_l1 = hop_chunks[1]
        part_r[rows(c_r1), :] = dot_r(c_r1).astype(jnp.bfloat16)
        part_l[rows(c_l1), :] = dot_l(c_l1).astype(jnp.bfloat16)
        c_r2, c_l2 = hop_chunks[2]
        part_r[rows(c_r2), :] = dot_r(c_r2).astype(jnp.bfloat16)
        part_l[rows(c_l2), :] = dot_l(c_l2).astype(jnp.bfloat16)
        part_r[rows(my), :] = dot_r(my).astype(jnp.bfloat16)
        part_l[rows(my), :] = dot_l(my).astype(jnp.bfloat16)

        for h in range(N_HOPS):
            for s in range(N_SUB):
                rdma_r, rdma_l = rdmas[(h, s)]
                if h < N_HOPS - 1:
                    nc_r, nc_l = hop_chunks[h + 1]
                    rdma_r.wait_recv()
                    part_r[sub_rows(nc_r, s), :] = (
                        part_r[sub_rows(nc_r, s), :].astype(jnp.float32)
                        + recv_r[sub_rows(h, s), :].astype(jnp.float32)
                    ).astype(jnp.bfloat16)
                    nxt = make_rdma(h + 1, s, nc_r, nc_l)
                    nxt[0].start()
                    rdma_l.wait_recv()
                    part_l[sub_rows(nc_l, s), :] = (
                        part_l[sub_rows(nc_l, s), :].astype(jnp.float32)
                        + recv_l[sub_rows(h, s), :].astype(jnp.float32)
                    ).astype(jnp.bfloat16)
                    nxt[1].start()
                    rdmas[(h + 1, s)] = nxt
                else:
                    rdma_r.wait_recv()
                    out_ref[pl.ds(s * SUB, SUB), :N_HALF] = (
                        recv_r[sub_rows(h, s), :].astype(jnp.float32)
                        + part_r[sub_rows(my, s), :].astype(jnp.float32))
                    rdma_l.wait_recv()
                    out_ref[pl.ds(s * SUB, SUB), N_HALF:] = (
                        recv_l[sub_rows(h, s), :].astype(jnp.float32)
                        + part_l[sub_rows(my, s), :].astype(jnp.float32))

        for rdma_r, rdma_l in rdmas.values():
            rdma_r.wait_send()
            rdma_l.wait_send()

    return pl.pallas_call(
        body,
        out_shape=jax.ShapeDtypeStruct((M_CHUNK, 2 * N_HALF), jnp.float32),
        in_specs=[pl.BlockSpec(memory_space=pltpu.VMEM),
                  pl.BlockSpec(memory_space=pltpu.VMEM)],
        out_specs=pl.BlockSpec(memory_space=pltpu.VMEM),
        scratch_shapes=[
            pltpu.VMEM((N_DEV * M_CHUNK, N_HALF), jnp.bfloat16),
            pltpu.VMEM((N_DEV * M_CHUNK, N_HALF), jnp.bfloat16),
            pltpu.VMEM((N_HOPS * M_CHUNK, N_HALF), jnp.bfloat16),
            pltpu.VMEM((N_HOPS * M_CHUNK, N_HALF), jnp.bfloat16),
            pltpu.SemaphoreType.DMA((N_HOPS, N_SUB)),
            pltpu.SemaphoreType.DMA((N_HOPS, N_SUB)),
            pltpu.SemaphoreType.DMA((N_HOPS, N_SUB)),
            pltpu.SemaphoreType.DMA((N_HOPS, N_SUB)),
        ],
        compiler_params=pltpu.CompilerParams(
            collective_id=0,
            vmem_limit_bytes=100 * 1024 * 1024,
        ),
    )(x, w_mat)


# device time: 92236 ns/iter; 1.2479x vs baseline; 1.0393x over previous
import jax
import jax.numpy as jnp
from jax import lax
from jax.experimental import pallas as pl
from jax.experimental.pallas import tpu as pltpu

N_DEV = 4
M_CHUNK = 1024
N_HALF = 1024
N_HOPS = N_DEV - 1
N_SUB = 4
SUB = M_CHUNK // N_SUB


def kernel(x, w_mat):
    def body(x_ref, w_ref, out_ref,
             part_r, part_l, recv_r, recv_l,
             send_r_sems, recv_r_sems, send_l_sems, recv_l_sems):
        my = lax.axis_index("i")
        right = (my + 1) % N_DEV
        left = (my + N_DEV - 1) % N_DEV

        barrier = pltpu.get_barrier_semaphore()
        for nbr in (left, right):
            pl.semaphore_signal(barrier, inc=1, device_id=(nbr,),
                                device_id_type=pl.DeviceIdType.MESH)
        pl.semaphore_wait(barrier, 2)

        def rows(c):
            return pl.ds(c * M_CHUNK, M_CHUNK)

        def sub_rows(c, s):
            return pl.ds(c * M_CHUNK + s * SUB, SUB)

        def dot_r(c):
            return jnp.dot(x_ref[rows(c), :], w_ref[:, :N_HALF],
                           preferred_element_type=jnp.float32)

        def dot_l(c):
            return jnp.dot(x_ref[rows(c), :], w_ref[:, N_HALF:],
                           preferred_element_type=jnp.float32)

        def dot_r_sub(c, s):
            return jnp.dot(x_ref[sub_rows(c, s), :], w_ref[:, :N_HALF],
                           preferred_element_type=jnp.float32)

        def dot_l_sub(c, s):
            return jnp.dot(x_ref[sub_rows(c, s), :], w_ref[:, N_HALF:],
                           preferred_element_type=jnp.float32)

        def make_rdma(h, s, c_r, c_l):
            rdma_r = pltpu.make_async_remote_copy(
                src_ref=part_r.at[sub_rows(c_r, s)],
                dst_ref=recv_r.at[sub_rows(h, s)],
                send_sem=send_r_sems.at[h, s],
                recv_sem=recv_r_sems.at[h, s],
                device_id=(right,),
                device_id_type=pl.DeviceIdType.MESH,
            )
            rdma_l = pltpu.make_async_remote_copy(
                src_ref=part_l.at[sub_rows(c_l, s)],
                dst_ref=recv_l.at[sub_rows(h, s)],
                send_sem=send_l_sems.at[h, s],
                recv_sem=recv_l_sems.at[h, s],
                device_id=(left,),
                device_id_type=pl.DeviceIdType.MESH,
            )
            return rdma_r, rdma_l

        hop_chunks = [((my + N_DEV - 1 - h) % N_DEV, (my + 1 + h) % N_DEV)
                      for h in range(N_HOPS)]

        c_r0, c_l0 = hop_chunks[0]
        rdmas = {}
        for s in range(N_SUB):
            part_r[sub_rows(c_r0, s), :] = dot_r_sub(c_r0, s).astype(jnp.bfloat16)
            part_l[sub_rows(c_l0, s), :] = dot_l_sub(c_l0, s).astype(jnp.bfloat16)
            rdmas[(0, s)] = make_rdma(0, s, c_r0, c_l0)
            rdmas[(0, s)][0].start()
            rdmas[(0, s)][1].start()

        c_r1, c_l1 = hop_chunks[1]
        part_r[rows(c_r1), :] = dot_r(c_r1).astype(jnp.bfloat16)
        part_l[rows(c_l1), :] = dot_l(c_l1).astype(jnp.bfloat16)
        c_r2, c_l2 = hop_chunks[2]
        part_r[rows(c_r2), :] = dot_r(c_r2).astype(jnp.bfloat16)
        part_l[rows(c_l2), :] = dot_l(c_l2).astype(jnp.bfloat16)
        part_r[rows(my), :] = dot_r(my).astype(jnp.bfloat16)
        part_l[rows(my), :] = dot_l(my).astype(jnp.bfloat16)

        for h in range(N_HOPS):
            for s in range(N_SUB):
                rdma_r, rdma_l = rdmas[(h, s)]
                if h < N_HOPS - 1:
                    nc_r, nc_l = hop_chunks[h + 1]
                    rdma_r.wait_recv()
                    part_r[sub_rows(nc_r, s), :] = (
                        part_r[sub_rows(nc_r, s), :]
                        + recv_r[sub_rows(h, s), :])
                    nxt = make_rdma(h + 1, s, nc_r, nc_l)
                    nxt[0].start()
                    rdma_l.wait_recv()
                    part_l[sub_rows(nc_l, s), :] = (
                        part_l[sub_rows(nc_l, s), :]
                        + recv_l[sub_rows(h, s), :])
                    nxt[1].start()
                    rdmas[(h + 1, s)] = nxt
                else:
                    rdma_r.wait_recv()
                    out_ref[pl.ds(s * SUB, SUB), :N_HALF] = (
                        recv_r[sub_rows(h, s), :].astype(jnp.float32)
                        + part_r[sub_rows(my, s), :].astype(jnp.float32))
                    rdma_l.wait_recv()
                    out_ref[pl.ds(s * SUB, SUB), N_HALF:] = (
                        recv_l[sub_rows(h, s), :].astype(jnp.float32)
                        + part_l[sub_rows(my, s), :].astype(jnp.float32))

        for rdma_r, rdma_l in rdmas.values():
            rdma_r.wait_send()
            rdma_l.wait_send()

    return pl.pallas_call(
        body,
        out_shape=jax.ShapeDtypeStruct((M_CHUNK, 2 * N_HALF), jnp.float32),
        in_specs=[pl.BlockSpec(memory_space=pltpu.VMEM),
                  pl.BlockSpec(memory_space=pltpu.VMEM)],
        out_specs=pl.BlockSpec(memory_space=pltpu.VMEM),
        scratch_shapes=[
            pltpu.VMEM((N_DEV * M_CHUNK, N_HALF), jnp.bfloat16),
            pltpu.VMEM((N_DEV * M_CHUNK, N_HALF), jnp.bfloat16),
            pltpu.VMEM((N_HOPS * M_CHUNK, N_HALF), jnp.bfloat16),
            pltpu.VMEM((N_HOPS * M_CHUNK, N_HALF), jnp.bfloat16),
            pltpu.SemaphoreType.DMA((N_HOPS, N_SUB)),
            pltpu.SemaphoreType.DMA((N_HOPS, N_SUB)),
            pltpu.SemaphoreType.DMA((N_HOPS, N_SUB)),
            pltpu.SemaphoreType.DMA((N_HOPS, N_SUB)),
        ],
        compiler_params=pltpu.CompilerParams(
            collective_id=0,
            vmem_limit_bytes=100 * 1024 * 1024,
        ),
    )(x, w_mat)
